# speedup vs baseline: 2.1402x; 2.1402x over previous
import functools

import jax
import jax.numpy as jnp
from jax import lax
from jax.experimental import pallas as pl
from jax.experimental.pallas import tpu as pltpu
from jax.experimental.pallas import tpu_sc as plsc

N = 10000
E = 160000
DIM = 256
G = 64
FH = DIM // 2

N_PAD = 10240
E_PAD = 163840
TRASH = N

NC, NS = 2, 16
CH = 128
E_PER_TILE = E_PAD // NS
N_CHUNKS = E_PER_TILE // CH
ROWS_PER_TILE = N_PAD // NS

BR = 256
NB = N_PAD // BR


def _agg_body(h0_hbm, h1_hbm, src_hbm, dst_hbm, out0_hbm, out1_hbm,
              src_v, dst_v, rows_v, acc_sh, gsem):
  c = lax.axis_index("c")
  s = lax.axis_index("s")
  r0 = s * ROWS_PER_TILE

  @pl.when(c == 0)
  def _():
    pltpu.sync_copy(h0_hbm.at[pl.ds(r0, ROWS_PER_TILE)],
                    acc_sh.at[pl.ds(r0, ROWS_PER_TILE)])

  @pl.when(c == 1)
  def _():
    pltpu.sync_copy(h1_hbm.at[pl.ds(r0, ROWS_PER_TILE)],
                    acc_sh.at[pl.ds(r0, ROWS_PER_TILE)])

  plsc.subcore_barrier()

  base = s * E_PER_TILE

  def body(i, carry):
    off = base + i * CH
    pltpu.sync_copy(src_hbm.at[pl.ds(off, CH)], src_v)
    pltpu.sync_copy(dst_hbm.at[pl.ds(off, CH)], dst_v)

    @pl.when(c == 0)
    def _():
      pltpu.async_copy(h0_hbm.at[src_v], rows_v, gsem).wait()

    @pl.when(c == 1)
    def _():
      pltpu.async_copy(h1_hbm.at[src_v], rows_v, gsem).wait()

    pltpu.sync_copy(rows_v, acc_sh.at[dst_v], add=True)
    return carry

  lax.fori_loop(0, N_CHUNKS, body, 0)
  plsc.subcore_barrier()

  @pl.when(c == 0)
  def _():
    pltpu.sync_copy(acc_sh.at[pl.ds(r0, ROWS_PER_TILE)],
                    out0_hbm.at[pl.ds(r0, ROWS_PER_TILE)])

  @pl.when(c == 1)
  def _():
    pltpu.sync_copy(acc_sh.at[pl.ds(r0, ROWS_PER_TILE)],
                    out1_hbm.at[pl.ds(r0, ROWS_PER_TILE)])


_agg_call = functools.partial(
    pl.kernel,
    out_type=(
        jax.ShapeDtypeStruct((N_PAD, FH), jnp.float32),
        jax.ShapeDtypeStruct((N_PAD, FH), jnp.float32),
    ),
    mesh=plsc.VectorSubcoreMesh(core_axis_name="c", subcore_axis_name="s"),
    scratch_types=[
        pltpu.VMEM((CH,), jnp.int32),
        pltpu.VMEM((CH,), jnp.int32),
        pltpu.VMEM((CH, FH), jnp.float32),
        pltpu.VMEM_SHARED((N_PAD, FH), jnp.float32),
        pltpu.SemaphoreType.DMA,
    ],
)(_agg_body)


def _mlp_body(u0, u1, w1, b1, w2, b2, z_ref, st_ref):
  i = pl.program_id(0)
  u = jnp.concatenate([u0[...], u1[...]], axis=1)
  a = jnp.dot(u, w1[...], preferred_element_type=jnp.float32) + b1[...]
  a = jnp.maximum(a, 0.0)
  z = jnp.dot(a, w2[...], preferred_element_type=jnp.float32) + b2[...]
  z = jnp.maximum(z, 0.0)
  row = i * BR + lax.broadcasted_iota(jnp.int32, (BR, DIM), 0)
  z = jnp.where(row < N, z, 0.0)
  z_ref[...] = z
  ssum = jnp.sum(z, axis=0, keepdims=True)
  ssq = jnp.sum(z * z, axis=0, keepdims=True)
  blk = jnp.concatenate(
      [ssum, ssq, jnp.zeros((6, DIM), jnp.float32)], axis=0)

  @pl.when(i == 0)
  def _():
    st_ref[...] = blk

  @pl.when(i > 0)
  def _():
    st_ref[...] = st_ref[...] + blk


_mlp_call = pl.pallas_call(
    _mlp_body,
    grid=(NB,),
    in_specs=[
        pl.BlockSpec((BR, FH), lambda i: (i, 0)),
        pl.BlockSpec((BR, FH), lambda i: (i, 0)),
        pl.BlockSpec((DIM, DIM), lambda i: (0, 0)),
        pl.BlockSpec((1, DIM), lambda i: (0, 0)),
        pl.BlockSpec((DIM, DIM), lambda i: (0, 0)),
        pl.BlockSpec((1, DIM), lambda i: (0, 0)),
    ],
    out_specs=[
        pl.BlockSpec((BR, DIM), lambda i: (i, 0)),
        pl.BlockSpec((8, DIM), lambda i: (0, 0)),
    ],
    out_shape=[
        jax.ShapeDtypeStruct((N_PAD, DIM), jnp.float32),
        jax.ShapeDtypeStruct((8, DIM), jnp.float32),
    ],
)


def _norm_body(z, st, gamma, beta, batchb, o0, o1, pooled):
  i = pl.program_id(0)
  mean = st[0:1, :] / N
  var = st[1:2, :] / N - mean * mean
  scale = gamma[...] / jnp.sqrt(var + 1e-5)
  shift = beta[...] - mean * scale
  out = z[...] * scale + shift
  o0[...] = out[:, :FH]
  o1[...] = out[:, FH:]
  b = batchb[0]
  oh = (b == lax.broadcasted_iota(jnp.int32, (G, BR), 0)).astype(jnp.float32)
  pc = jnp.dot(oh, out, preferred_element_type=jnp.float32)

  @pl.when(i == 0)
  def _():
    pooled[...] = pc

  @pl.when(i > 0)
  def _():
    pooled[...] = pooled[...] + pc


_norm_call = pl.pallas_call(
    _norm_body,
    grid=(NB,),
    in_specs=[
        pl.BlockSpec((BR, DIM), lambda i: (i, 0)),
        pl.BlockSpec((8, DIM), lambda i: (0, 0)),
        pl.BlockSpec((1, DIM), lambda i: (0, 0)),
        pl.BlockSpec((1, DIM), lambda i: (0, 0)),
        pl.BlockSpec((1, 1, BR), lambda i: (i, 0, 0)),
    ],
    out_specs=[
        pl.BlockSpec((BR, FH), lambda i: (i, 0)),
        pl.BlockSpec((BR, FH), lambda i: (i, 0)),
        pl.BlockSpec((G, DIM), lambda i: (0, 0)),
    ],
    out_shape=[
        jax.ShapeDtypeStruct((N_PAD, FH), jnp.float32),
        jax.ShapeDtypeStruct((N_PAD, FH), jnp.float32),
        jax.ShapeDtypeStruct((G, DIM), jnp.float32),
    ],
)


def kernel(x, edge_index, batch, params):
  src = edge_index[0]
  dst = edge_index[1]
  src_p = jnp.concatenate(
      [src, jnp.zeros((E_PAD - E,), jnp.int32)])
  dst_p = jnp.concatenate(
      [dst, jnp.full((E_PAD - E,), TRASH, jnp.int32)])
  batch_p = jnp.concatenate(
      [batch, jnp.full((N_PAD - N,), G, jnp.int32)]).reshape(NB, 1, BR)

  xp = jnp.pad(x, ((0, N_PAD - N), (0, 0)))
  h0 = xp[:, :FH]
  h1 = xp[:, FH:]

  pooled_list = []
  xs_list = []
  for p in params:
    u0, u1 = _agg_call(h0, h1, src_p, dst_p)
    z, st = _mlp_call(u0, u1, p["W1"], p["b1"].reshape(1, DIM),
                      p["W2"], p["b2"].reshape(1, DIM))
    o0, o1, pooled = _norm_call(z, st, p["gamma"].reshape(1, DIM),
                                p["beta"].reshape(1, DIM), batch_p)
    h0, h1 = o0, o1
    xs_list.append(jnp.concatenate([o0[:N], o1[:N]], axis=1))
    pooled_list.append(pooled)

  return (jnp.concatenate(pooled_list, axis=1),
          jnp.concatenate(xs_list, axis=1))


# speedup vs baseline: 2.9398x; 1.3736x over previous
import functools

import jax
import jax.numpy as jnp
from jax import lax
from jax.experimental import pallas as pl
from jax.experimental.pallas import tpu as pltpu
from jax.experimental.pallas import tpu_sc as plsc

N = 10000
E = 160000
DIM = 256
G = 64
FH = DIM // 2

N_PAD = 10240
E_PAD = 163840
TRASH = N

NC, NS = 2, 16
CH = 64
E_PER_TILE = E_PAD // NS
N_CHUNKS = E_PER_TILE // CH
NBUF = 5
D1 = 2
D2 = 2
ROWS_PER_TILE = N_PAD // NS

BR = 256
NB = N_PAD // BR


def _agg_body(h0_hbm, h1_hbm, eidx_hbm, out0_hbm, out1_hbm,
              idx32, rows_v, acc_sh, *sems):
  isems = sems[:NBUF]
  gsems = sems[NBUF:2 * NBUF]
  ssems = sems[2 * NBUF:]
  c = lax.axis_index("c")
  s = lax.axis_index("s")
  r0 = s * ROWS_PER_TILE

  @pl.when(c == 0)
  def _():
    pltpu.sync_copy(h0_hbm.at[pl.ds(r0, ROWS_PER_TILE)],
                    acc_sh.at[pl.ds(r0, ROWS_PER_TILE)])

  @pl.when(c == 1)
  def _():
    pltpu.sync_copy(h1_hbm.at[pl.ds(r0, ROWS_PER_TILE)],
                    acc_sh.at[pl.ds(r0, ROWS_PER_TILE)])

  plsc.subcore_barrier()

  def iload_start(j, k):
    pltpu.async_copy(eidx_hbm.at[s, j], idx32.at[k], isems[k])

  def iload_wait(j, k):
    pltpu.make_async_copy(eidx_hbm.at[s, j], idx32.at[k], isems[k]).wait()

  def gather_start(k):
    @pl.when(c == 0)
    def _():
      pltpu.async_copy(h0_hbm.at[idx32.at[k, 0]], rows_v.at[k], gsems[k])

    @pl.when(c == 1)
    def _():
      pltpu.async_copy(h1_hbm.at[idx32.at[k, 0]], rows_v.at[k], gsems[k])

  def gather_wait(k):
    @pl.when(c == 0)
    def _():
      pltpu.make_async_copy(h0_hbm.at[idx32.at[k, 0]], rows_v.at[k],
                            gsems[k]).wait()

    @pl.when(c == 1)
    def _():
      pltpu.make_async_copy(h1_hbm.at[idx32.at[k, 0]], rows_v.at[k],
                            gsems[k]).wait()

  def scatter_start(k):
    pltpu.async_copy(rows_v.at[k], acc_sh.at[idx32.at[k, 1]], ssems[k],
                     add=True)

  def scatter_wait(k):
    pltpu.make_async_copy(rows_v.at[k], acc_sh.at[idx32.at[k, 1]],
                          ssems[k]).wait()

  n_outer = (N_CHUNKS + D1 + D2 + NBUF - 1) // NBUF

  @pl.loop(0, n_outer)
  def _(t):
    p0 = t * NBUF
    for kk in range(NBUF):
      ji = p0 + kk
      jg = ji - D1
      js = ji - D1 - D2
      kg = (kk - D1) % NBUF
      ks = (kk - D1 - D2) % NBUF

      @pl.when(ji < N_CHUNKS)
      def _():
        @pl.when(ji >= NBUF)
        def _():
          scatter_wait(kk)

        iload_start(ji, kk)

      @pl.when(jnp.logical_and(jg >= 0, jg < N_CHUNKS))
      def _():
        iload_wait(jg, kg)
        gather_start(kg)

      @pl.when(jnp.logical_and(js >= 0, js < N_CHUNKS))
      def _():
        gather_wait(ks)
        scatter_start(ks)

  for k in range(NBUF):
    scatter_wait(k)

  plsc.subcore_barrier()

  @pl.when(c == 0)
  def _():
    pltpu.sync_copy(acc_sh.at[pl.ds(r0, ROWS_PER_TILE)],
                    out0_hbm.at[pl.ds(r0, ROWS_PER_TILE)])

  @pl.when(c == 1)
  def _():
    pltpu.sync_copy(acc_sh.at[pl.ds(r0, ROWS_PER_TILE)],
                    out1_hbm.at[pl.ds(r0, ROWS_PER_TILE)])


_agg_call = functools.partial(
    pl.kernel,
    out_type=(
        jax.ShapeDtypeStruct((N_PAD, FH), jnp.float32),
        jax.ShapeDtypeStruct((N_PAD, FH), jnp.float32),
    ),
    mesh=plsc.VectorSubcoreMesh(core_axis_name="c", subcore_axis_name="s"),
    scratch_types=[
        pltpu.VMEM((NBUF, 2, CH), jnp.int32),
        pltpu.VMEM((NBUF, CH, FH), jnp.float32),
        pltpu.VMEM_SHARED((N_PAD, FH), jnp.float32),
    ] + [pltpu.SemaphoreType.DMA] * (3 * NBUF),
)(_agg_body)


def _mlp_body(u0, u1, w1, b1, w2, b2, z_ref, st_ref):
  i = pl.program_id(0)
  u = jnp.concatenate([u0[...], u1[...]], axis=1)
  a = jnp.dot(u, w1[...], preferred_element_type=jnp.float32) + b1[...]
  a = jnp.maximum(a, 0.0)
  z = jnp.dot(a, w2[...], preferred_element_type=jnp.float32) + b2[...]
  z = jnp.maximum(z, 0.0)
  row = i * BR + lax.broadcasted_iota(jnp.int32, (BR, DIM), 0)
  z = jnp.where(row < N, z, 0.0)
  z_ref[...] = z
  ssum = jnp.sum(z, axis=0, keepdims=True)
  ssq = jnp.sum(z * z, axis=0, keepdims=True)
  blk = jnp.concatenate(
      [ssum, ssq, jnp.zeros((6, DIM), jnp.float32)], axis=0)

  @pl.when(i == 0)
  def _():
    st_ref[...] = blk

  @pl.when(i > 0)
  def _():
    st_ref[...] = st_ref[...] + blk


_mlp_call = pl.pallas_call(
    _mlp_body,
    grid=(NB,),
    in_specs=[
        pl.BlockSpec((BR, FH), lambda i: (i, 0)),
        pl.BlockSpec((BR, FH), lambda i: (i, 0)),
        pl.BlockSpec((DIM, DIM), lambda i: (0, 0)),
        pl.BlockSpec((1, DIM), lambda i: (0, 0)),
        pl.BlockSpec((DIM, DIM), lambda i: (0, 0)),
        pl.BlockSpec((1, DIM), lambda i: (0, 0)),
    ],
    out_specs=[
        pl.BlockSpec((BR, DIM), lambda i: (i, 0)),
        pl.BlockSpec((8, DIM), lambda i: (0, 0)),
    ],
    out_shape=[
        jax.ShapeDtypeStruct((N_PAD, DIM), jnp.float32),
        jax.ShapeDtypeStruct((8, DIM), jnp.float32),
    ],
)


def _norm_body(z, st, gamma, beta, batchb, o0, o1, pooled):
  i = pl.program_id(0)
  mean = st[0:1, :] / N
  var = st[1:2, :] / N - mean * mean
  scale = gamma[...] / jnp.sqrt(var + 1e-5)
  shift = beta[...] - mean * scale
  out = z[...] * scale + shift
  o0[...] = out[:, :FH]
  o1[...] = out[:, FH:]
  b = batchb[0]
  oh = (b == lax.broadcasted_iota(jnp.int32, (G, BR), 0)).astype(jnp.float32)
  pc = jnp.dot(oh, out, preferred_element_type=jnp.float32)

  @pl.when(i == 0)
  def _():
    pooled[...] = pc

  @pl.when(i > 0)
  def _():
    pooled[...] = pooled[...] + pc


_norm_call = pl.pallas_call(
    _norm_body,
    grid=(NB,),
    in_specs=[
        pl.BlockSpec((BR, DIM), lambda i: (i, 0)),
        pl.BlockSpec((8, DIM), lambda i: (0, 0)),
        pl.BlockSpec((1, DIM), lambda i: (0, 0)),
        pl.BlockSpec((1, DIM), lambda i: (0, 0)),
        pl.BlockSpec((1, 1, BR), lambda i: (i, 0, 0)),
    ],
    out_specs=[
        pl.BlockSpec((BR, FH), lambda i: (i, 0)),
        pl.BlockSpec((BR, FH), lambda i: (i, 0)),
        pl.BlockSpec((G, DIM), lambda i: (0, 0)),
    ],
    out_shape=[
        jax.ShapeDtypeStruct((N_PAD, FH), jnp.float32),
        jax.ShapeDtypeStruct((N_PAD, FH), jnp.float32),
        jax.ShapeDtypeStruct((G, DIM), jnp.float32),
    ],
)


def kernel(x, edge_index, batch, params):
  src = edge_index[0]
  dst = edge_index[1]
  src_p = jnp.concatenate([src, jnp.zeros((E_PAD - E,), jnp.int32)])
  dst_p = jnp.concatenate([dst, jnp.full((E_PAD - E,), TRASH, jnp.int32)])
  eidx = jnp.concatenate(
      [src_p.reshape(NS, N_CHUNKS, 1, CH),
       dst_p.reshape(NS, N_CHUNKS, 1, CH)], axis=2)
  batch_p = jnp.concatenate(
      [batch, jnp.full((N_PAD - N,), G, jnp.int32)]).reshape(NB, 1, BR)

  xp = jnp.pad(x, ((0, N_PAD - N), (0, 0)))
  h0 = xp[:, :FH]
  h1 = xp[:, FH:]

  pooled_list = []
  xs_list = []
  for p in params:
    u0, u1 = _agg_call(h0, h1, eidx)
    z, st = _mlp_call(u0, u1, p["W1"], p["b1"].reshape(1, DIM),
                      p["W2"], p["b2"].reshape(1, DIM))
    o0, o1, pooled = _norm_call(z, st, p["gamma"].reshape(1, DIM),
                                p["beta"].reshape(1, DIM), batch_p)
    h0, h1 = o0, o1
    xs_list.append(jnp.concatenate([o0[:N], o1[:N]], axis=1))
    pooled_list.append(pooled)

  return (jnp.concatenate(pooled_list, axis=1),
          jnp.concatenate(xs_list, axis=1))


# speedup vs baseline: 6.5116x; 2.2149x over previous
import functools

import jax
import jax.numpy as jnp
from jax import lax
from jax.experimental import pallas as pl
from jax.experimental.pallas import tpu as pltpu
from jax.experimental.pallas import tpu_sc as plsc

N = 10000
E = 160000
DIM = 256
G = 64
FH = DIM // 2

N_PAD = 10240

NC, NS = 2, 16
CH = 64
N_CHUNKS = E // CH
CH_BASE = N_CHUNKS // NS
CH_EXTRA = N_CHUNKS % NS
NBUF = 6
D1 = 2
D2 = 2
ROWS_PER_TILE = 624
LAST_ROWS = N - ROWS_PER_TILE * (NS - 1)
N_ACC = N

BR = 256
NB = N_PAD // BR


def _agg_body(h0_hbm, h1_hbm, srcc_hbm, dstc_hbm, out0_hbm, out1_hbm,
              src32, dst32, rows_v, acc_sh, *sems):
  isems = sems[:NBUF]
  gsems = sems[NBUF:2 * NBUF]
  ssems = sems[2 * NBUF:]
  c = lax.axis_index("c")
  s = lax.axis_index("s")
  r0 = s * ROWS_PER_TILE
  ts = s * CH_BASE + jnp.minimum(s, CH_EXTRA)
  nch = CH_BASE + jnp.where(s < CH_EXTRA, 1, 0)

  def stripe_copy(src_ref, dst_ref):
    @pl.when(s < NS - 1)
    def _():
      pltpu.sync_copy(src_ref.at[pl.ds(r0, ROWS_PER_TILE)],
                      dst_ref.at[pl.ds(r0, ROWS_PER_TILE)])

    @pl.when(s == NS - 1)
    def _():
      pltpu.sync_copy(src_ref.at[pl.ds(r0, LAST_ROWS)],
                      dst_ref.at[pl.ds(r0, LAST_ROWS)])

  @pl.when(c == 0)
  def _():
    stripe_copy(h0_hbm, acc_sh)

  @pl.when(c == 1)
  def _():
    stripe_copy(h1_hbm, acc_sh)

  plsc.subcore_barrier()

  def iload_start(j, k):
    pltpu.async_copy(srcc_hbm.at[pl.ds(ts + j, 1)], src32.at[pl.ds(k, 1)],
                     isems[k])
    pltpu.async_copy(dstc_hbm.at[pl.ds(ts + j, 1)], dst32.at[pl.ds(k, 1)],
                     isems[k])

  def iload_wait(j, k):
    pltpu.make_async_copy(srcc_hbm.at[pl.ds(ts + j, 1)],
                          src32.at[pl.ds(k, 1)], isems[k]).wait()
    pltpu.make_async_copy(dstc_hbm.at[pl.ds(ts + j, 1)],
                          dst32.at[pl.ds(k, 1)], isems[k]).wait()

  def gather_start(k):
    @pl.when(c == 0)
    def _():
      pltpu.async_copy(h0_hbm.at[src32.at[k, 0]], rows_v.at[k], gsems[k])

    @pl.when(c == 1)
    def _():
      pltpu.async_copy(h1_hbm.at[src32.at[k, 0]], rows_v.at[k], gsems[k])

  def gather_wait(k):
    @pl.when(c == 0)
    def _():
      pltpu.make_async_copy(h0_hbm.at[src32.at[k, 0]], rows_v.at[k],
                            gsems[k]).wait()

    @pl.when(c == 1)
    def _():
      pltpu.make_async_copy(h1_hbm.at[src32.at[k, 0]], rows_v.at[k],
                            gsems[k]).wait()

  def scatter_start(k):
    pltpu.async_copy(rows_v.at[k], acc_sh.at[dst32.at[k, 0]], ssems[k],
                     add=True)

  def scatter_wait(k):
    pltpu.make_async_copy(rows_v.at[k], acc_sh.at[dst32.at[k, 0]],
                          ssems[k]).wait()

  n_outer = (nch + D1 + D2 + NBUF - 1) // NBUF

  @pl.loop(0, n_outer)
  def _(t):
    p0 = t * NBUF
    for kk in range(NBUF):
      ji = p0 + kk
      jg = ji - D1
      js = ji - D1 - D2
      kg = (kk - D1) % NBUF
      ks = (kk - D1 - D2) % NBUF

      @pl.when(ji < nch)
      def _():
        @pl.when(ji >= NBUF)
        def _():
          scatter_wait(kk)

        iload_start(ji, kk)

      @pl.when(jnp.logical_and(jg >= 0, jg < nch))
      def _():
        iload_wait(jg, kg)
        gather_start(kg)

      @pl.when(jnp.logical_and(js >= 0, js < nch))
      def _():
        gather_wait(ks)
        scatter_start(ks)

  for k in range(NBUF):
    scatter_wait(k)

  plsc.subcore_barrier()

  @pl.when(c == 0)
  def _():
    stripe_copy(acc_sh, out0_hbm)

  @pl.when(c == 1)
  def _():
    stripe_copy(acc_sh, out1_hbm)


_agg_call = functools.partial(
    pl.kernel,
    out_type=(
        jax.ShapeDtypeStruct((N_PAD, FH), jnp.float32),
        jax.ShapeDtypeStruct((N_PAD, FH), jnp.float32),
    ),
    mesh=plsc.VectorSubcoreMesh(core_axis_name="c", subcore_axis_name="s"),
    scratch_types=[
        pltpu.VMEM((NBUF, 1, CH), jnp.int32),
        pltpu.VMEM((NBUF, 1, CH), jnp.int32),
        pltpu.VMEM((NBUF, CH, FH), jnp.float32),
        pltpu.VMEM_SHARED((N_ACC, FH), jnp.float32),
    ] + [pltpu.SemaphoreType.DMA] * (3 * NBUF),
)(_agg_body)


def _mlp_body(u0, u1, w1, b1, w2, b2, z_ref, st_ref):
  i = pl.program_id(0)
  u = jnp.concatenate([u0[...], u1[...]], axis=1)
  a = jnp.dot(u, w1[...], preferred_element_type=jnp.float32) + b1[...]
  a = jnp.maximum(a, 0.0)
  z = jnp.dot(a, w2[...], preferred_element_type=jnp.float32) + b2[...]
  z = jnp.maximum(z, 0.0)
  row = i * BR + lax.broadcasted_iota(jnp.int32, (BR, DIM), 0)
  z = jnp.where(row < N, z, 0.0)
  z_ref[...] = z
  ssum = jnp.sum(z, axis=0, keepdims=True)
  ssq = jnp.sum(z * z, axis=0, keepdims=True)
  blk = jnp.concatenate(
      [ssum, ssq, jnp.zeros((6, DIM), jnp.float32)], axis=0)

  @pl.when(i == 0)
  def _():
    st_ref[...] = blk

  @pl.when(i > 0)
  def _():
    st_ref[...] = st_ref[...] + blk


_mlp_call = pl.pallas_call(
    _mlp_body,
    grid=(NB,),
    in_specs=[
        pl.BlockSpec((BR, FH), lambda i: (i, 0)),
        pl.BlockSpec((BR, FH), lambda i: (i, 0)),
        pl.BlockSpec((DIM, DIM), lambda i: (0, 0)),
        pl.BlockSpec((1, DIM), lambda i: (0, 0)),
        pl.BlockSpec((DIM, DIM), lambda i: (0, 0)),
        pl.BlockSpec((1, DIM), lambda i: (0, 0)),
    ],
    out_specs=[
        pl.BlockSpec((BR, DIM), lambda i: (i, 0)),
        pl.BlockSpec((8, DIM), lambda i: (0, 0)),
    ],
    out_shape=[
        jax.ShapeDtypeStruct((N_PAD, DIM), jnp.float32),
        jax.ShapeDtypeStruct((8, DIM), jnp.float32),
    ],
)


def _norm_body(z, st, gamma, beta, batchb, o0, o1, pooled):
  i = pl.program_id(0)
  mean = st[0:1, :] / N
  var = st[1:2, :] / N - mean * mean
  scale = gamma[...] / jnp.sqrt(var + 1e-5)
  shift = beta[...] - mean * scale
  out = z[...] * scale + shift
  o0[...] = out[:, :FH]
  o1[...] = out[:, FH:]
  b = batchb[0]
  oh = (b == lax.broadcasted_iota(jnp.int32, (G, BR), 0)).astype(jnp.float32)
  pc = jnp.dot(oh, out, preferred_element_type=jnp.float32)

  @pl.when(i == 0)
  def _():
    pooled[...] = pc

  @pl.when(i > 0)
  def _():
    pooled[...] = pooled[...] + pc


_norm_call = pl.pallas_call(
    _norm_body,
    grid=(NB,),
    in_specs=[
        pl.BlockSpec((BR, DIM), lambda i: (i, 0)),
        pl.BlockSpec((8, DIM), lambda i: (0, 0)),
        pl.BlockSpec((1, DIM), lambda i: (0, 0)),
        pl.BlockSpec((1, DIM), lambda i: (0, 0)),
        pl.BlockSpec((1, 1, BR), lambda i: (i, 0, 0)),
    ],
    out_specs=[
        pl.BlockSpec((BR, FH), lambda i: (i, 0)),
        pl.BlockSpec((BR, FH), lambda i: (i, 0)),
        pl.BlockSpec((G, DIM), lambda i: (0, 0)),
    ],
    out_shape=[
        jax.ShapeDtypeStruct((N_PAD, FH), jnp.float32),
        jax.ShapeDtypeStruct((N_PAD, FH), jnp.float32),
        jax.ShapeDtypeStruct((G, DIM), jnp.float32),
    ],
)


def kernel(x, edge_index, batch, params):
  src = edge_index[0]
  dst = edge_index[1]
  srcc = src.reshape(N_CHUNKS, 1, CH)
  dstc = dst.reshape(N_CHUNKS, 1, CH)
  batch_p = jnp.concatenate(
      [batch, jnp.full((N_PAD - N,), G, jnp.int32)]).reshape(NB, 1, BR)

  xp = jnp.pad(x, ((0, N_PAD - N), (0, 0)))
  h0 = xp[:, :FH]
  h1 = xp[:, FH:]

  pooled_list = []
  xs_list = []
  for p in params:
    u0, u1 = _agg_call(h0, h1, srcc, dstc)
    z, st = _mlp_call(u0, u1, p["W1"], p["b1"].reshape(1, DIM),
                      p["W2"], p["b2"].reshape(1, DIM))
    o0, o1, pooled = _norm_call(z, st, p["gamma"].reshape(1, DIM),
                                p["beta"].reshape(1, DIM), batch_p)
    h0, h1 = o0, o1
    xs_list.append(jnp.concatenate([o0[:N], o1[:N]], axis=1))
    pooled_list.append(pooled)

  return (jnp.concatenate(pooled_list, axis=1),
          jnp.concatenate(xs_list, axis=1))


# speedup vs baseline: 6.5574x; 1.0070x over previous
import functools

import jax
import jax.numpy as jnp
from jax import lax
from jax.experimental import pallas as pl
from jax.experimental.pallas import tpu as pltpu
from jax.experimental.pallas import tpu_sc as plsc

N = 10000
E = 160000
DIM = 256
G = 64
FH = DIM // 2

N_PAD = 10240

NC, NS = 2, 16
CH = 64
N_CHUNKS = E // CH
CH_BASE = N_CHUNKS // NS
CH_EXTRA = N_CHUNKS % NS
NBUF = 6
D1 = 2
D2 = 2
ROWS_PER_TILE = 624
LAST_ROWS = N - ROWS_PER_TILE * (NS - 1)
N_ACC = N

BR = 256
NB = N_PAD // BR


def _agg_body(h0_hbm, h1_hbm, srcc_hbm, dstc_hbm, out0_hbm, out1_hbm,
              src32, dst32, rows_v, acc_sh, *sems):
  isems = sems[:NBUF]
  gsems = sems[NBUF:2 * NBUF]
  ssems = sems[2 * NBUF:]
  c = lax.axis_index("c")
  s = lax.axis_index("s")
  r0 = s * ROWS_PER_TILE
  ts = s * CH_BASE + jnp.minimum(s, CH_EXTRA)
  nch = CH_BASE + jnp.where(s < CH_EXTRA, 1, 0)

  def stripe_copy(src_ref, dst_ref):
    @pl.when(s < NS - 1)
    def _():
      pltpu.sync_copy(src_ref.at[pl.ds(r0, ROWS_PER_TILE)],
                      dst_ref.at[pl.ds(r0, ROWS_PER_TILE)])

    @pl.when(s == NS - 1)
    def _():
      pltpu.sync_copy(src_ref.at[pl.ds(r0, LAST_ROWS)],
                      dst_ref.at[pl.ds(r0, LAST_ROWS)])

  @pl.when(c == 0)
  def _():
    stripe_copy(h0_hbm, acc_sh)

  @pl.when(c == 1)
  def _():
    stripe_copy(h1_hbm, acc_sh)

  plsc.subcore_barrier()

  def iload_start(j, k):
    pltpu.async_copy(srcc_hbm.at[pl.ds(ts + j, 1)], src32.at[pl.ds(k, 1)],
                     isems[k])
    pltpu.async_copy(dstc_hbm.at[pl.ds(ts + j, 1)], dst32.at[pl.ds(k, 1)],
                     isems[k])

  def iload_wait(j, k):
    pltpu.make_async_copy(srcc_hbm.at[pl.ds(ts + j, 1)],
                          src32.at[pl.ds(k, 1)], isems[k]).wait()
    pltpu.make_async_copy(dstc_hbm.at[pl.ds(ts + j, 1)],
                          dst32.at[pl.ds(k, 1)], isems[k]).wait()

  def gather_start(k):
    @pl.when(c == 0)
    def _():
      pltpu.async_copy(h0_hbm.at[src32.at[k, 0]], rows_v.at[k], gsems[k])

    @pl.when(c == 1)
    def _():
      pltpu.async_copy(h1_hbm.at[src32.at[k, 0]], rows_v.at[k], gsems[k])

  def gather_wait(k):
    @pl.when(c == 0)
    def _():
      pltpu.make_async_copy(h0_hbm.at[src32.at[k, 0]], rows_v.at[k],
                            gsems[k]).wait()

    @pl.when(c == 1)
    def _():
      pltpu.make_async_copy(h1_hbm.at[src32.at[k, 0]], rows_v.at[k],
                            gsems[k]).wait()

  def scatter_start(k):
    pltpu.async_copy(rows_v.at[k], acc_sh.at[dst32.at[k, 0]], ssems[k],
                     add=True)

  def scatter_wait(k):
    pltpu.make_async_copy(rows_v.at[k], acc_sh.at[dst32.at[k, 0]],
                          ssems[k]).wait()

  n_outer = (nch + D1 + D2 + NBUF - 1) // NBUF

  @pl.loop(0, n_outer)
  def _(t):
    p0 = t * NBUF
    for kk in range(NBUF):
      ji = p0 + kk
      jg = ji - D1
      js = ji - D1 - D2
      kg = (kk - D1) % NBUF
      ks = (kk - D1 - D2) % NBUF

      @pl.when(ji < nch)
      def _():
        @pl.when(ji >= NBUF)
        def _():
          scatter_wait(kk)

        iload_start(ji, kk)

      @pl.when(jnp.logical_and(jg >= 0, jg < nch))
      def _():
        iload_wait(jg, kg)
        gather_start(kg)

      @pl.when(jnp.logical_and(js >= 0, js < nch))
      def _():
        gather_wait(ks)
        scatter_start(ks)

  for k in range(NBUF):
    scatter_wait(k)

  plsc.subcore_barrier()

  @pl.when(c == 0)
  def _():
    stripe_copy(acc_sh, out0_hbm)

  @pl.when(c == 1)
  def _():
    stripe_copy(acc_sh, out1_hbm)


_agg_call = functools.partial(
    pl.kernel,
    out_type=(
        jax.ShapeDtypeStruct((N_PAD, FH), jnp.float32),
        jax.ShapeDtypeStruct((N_PAD, FH), jnp.float32),
    ),
    mesh=plsc.VectorSubcoreMesh(core_axis_name="c", subcore_axis_name="s"),
    scratch_types=[
        pltpu.VMEM((NBUF, 1, CH), jnp.int32),
        pltpu.VMEM((NBUF, 1, CH), jnp.int32),
        pltpu.VMEM((NBUF, CH, FH), jnp.float32),
        pltpu.VMEM_SHARED((N_ACC, FH), jnp.float32),
    ] + [pltpu.SemaphoreType.DMA] * (3 * NBUF),
)(_agg_body)


def _layer_body(u0, u1, w1, b1, w2, b2, gamma, beta, batchb,
                o0, o1, pooled, z_s, ssum_s, ssq_s):
  p = pl.program_id(0)
  i = pl.program_id(1)

  @pl.when(p == 0)
  def _():
    u = jnp.concatenate([u0[...], u1[...]], axis=1)
    a = jnp.dot(u, w1[...], preferred_element_type=jnp.float32) + b1[...]
    a = jnp.maximum(a, 0.0)
    z = jnp.dot(a, w2[...], preferred_element_type=jnp.float32) + b2[...]
    z = jnp.maximum(z, 0.0)
    row = i * BR + lax.broadcasted_iota(jnp.int32, (BR, DIM), 0)
    z = jnp.where(row < N, z, 0.0)
    z_s[pl.ds(i * BR, BR), :] = z
    ssum = jnp.sum(z, axis=0, keepdims=True)
    ssq = jnp.sum(z * z, axis=0, keepdims=True)

    @pl.when(i == 0)
    def _():
      ssum_s[...] = ssum
      ssq_s[...] = ssq

    @pl.when(i > 0)
    def _():
      ssum_s[...] = ssum_s[...] + ssum
      ssq_s[...] = ssq_s[...] + ssq

  @pl.when(p == 1)
  def _():
    mean = ssum_s[...] / N
    var = ssq_s[...] / N - mean * mean
    scale = gamma[...] / jnp.sqrt(var + 1e-5)
    shift = beta[...] - mean * scale
    out = z_s[pl.ds(i * BR, BR), :] * scale + shift
    o0[...] = out[:, :FH]
    o1[...] = out[:, FH:]
    oh = (batchb[0] == lax.broadcasted_iota(jnp.int32, (G, BR), 0)
          ).astype(jnp.float32)
    pc = jnp.dot(oh, out, preferred_element_type=jnp.float32)

    @pl.when(i == 0)
    def _():
      pooled[...] = pc

    @pl.when(i > 0)
    def _():
      pooled[...] = pooled[...] + pc


_layer_call = pl.pallas_call(
    _layer_body,
    grid=(2, NB),
    in_specs=[
        pl.BlockSpec((BR, FH), lambda p, i: (i, 0)),
        pl.BlockSpec((BR, FH), lambda p, i: (i, 0)),
        pl.BlockSpec((DIM, DIM), lambda p, i: (0, 0)),
        pl.BlockSpec((1, DIM), lambda p, i: (0, 0)),
        pl.BlockSpec((DIM, DIM), lambda p, i: (0, 0)),
        pl.BlockSpec((1, DIM), lambda p, i: (0, 0)),
        pl.BlockSpec((1, DIM), lambda p, i: (0, 0)),
        pl.BlockSpec((1, DIM), lambda p, i: (0, 0)),
        pl.BlockSpec((1, 1, BR), lambda p, i: (i, 0, 0)),
    ],
    out_specs=[
        pl.BlockSpec((BR, FH), lambda p, i: (i, 0)),
        pl.BlockSpec((BR, FH), lambda p, i: (i, 0)),
        pl.BlockSpec((G, DIM), lambda p, i: (0, 0)),
    ],
    out_shape=[
        jax.ShapeDtypeStruct((N_PAD, FH), jnp.float32),
        jax.ShapeDtypeStruct((N_PAD, FH), jnp.float32),
        jax.ShapeDtypeStruct((G, DIM), jnp.float32),
    ],
    scratch_shapes=[
        pltpu.VMEM((N_PAD, DIM), jnp.float32),
        pltpu.VMEM((1, DIM), jnp.float32),
        pltpu.VMEM((1, DIM), jnp.float32),
    ],
)


def kernel(x, edge_index, batch, params):
  src = edge_index[0]
  dst = edge_index[1]
  srcc = src.reshape(N_CHUNKS, 1, CH)
  dstc = dst.reshape(N_CHUNKS, 1, CH)
  batch_p = jnp.concatenate(
      [batch, jnp.full((N_PAD - N,), G, jnp.int32)]).reshape(NB, 1, BR)

  xp = jnp.pad(x, ((0, N_PAD - N), (0, 0)))
  h0 = xp[:, :FH]
  h1 = xp[:, FH:]

  pooled_list = []
  xs_list = []
  for p in params:
    u0, u1 = _agg_call(h0, h1, srcc, dstc)
    o0, o1, pooled = _layer_call(
        u0, u1, p["W1"], p["b1"].reshape(1, DIM),
        p["W2"], p["b2"].reshape(1, DIM), p["gamma"].reshape(1, DIM),
        p["beta"].reshape(1, DIM), batch_p)
    h0, h1 = o0, o1
    xs_list.append(jnp.concatenate([o0[:N], o1[:N]], axis=1))
    pooled_list.append(pooled)

  return (jnp.concatenate(pooled_list, axis=1),
          jnp.concatenate(xs_list, axis=1))


# speedup vs baseline: 7.5854x; 1.1568x over previous
import functools

import jax
import jax.numpy as jnp
from jax import lax
from jax.experimental import pallas as pl
from jax.experimental.pallas import tpu as pltpu
from jax.experimental.pallas import tpu_sc as plsc

N = 10000
E = 160000
DIM = 256
G = 64
FH = DIM // 2

N_PAD = 10240

NC, NS = 2, 16
CH = 64
N_CHUNKS = E // CH
CH_BASE = N_CHUNKS // NS
CH_EXTRA = N_CHUNKS % NS
NBUF = 6
D1 = 1
D2 = 4
ROWS_PER_TILE = 624
LAST_ROWS = N - ROWS_PER_TILE * (NS - 1)
N_ACC = N

BR = 512
NB = N_PAD // BR


def _agg_body(h0_hbm, h1_hbm, srcc_hbm, dstc_hbm, out0_hbm, out1_hbm,
              src32, dst32, rows_v, acc_sh, *sems):
  isems = sems[:NBUF]
  gsems = sems[NBUF:2 * NBUF]
  ssems = sems[2 * NBUF:]
  c = lax.axis_index("c")
  s = lax.axis_index("s")
  r0 = s * ROWS_PER_TILE
  ts = s * CH_BASE + jnp.minimum(s, CH_EXTRA)
  nch = CH_BASE + jnp.where(s < CH_EXTRA, 1, 0)

  def stripe_copy(src_ref, dst_ref):
    @pl.when(s < NS - 1)
    def _():
      pltpu.sync_copy(src_ref.at[pl.ds(r0, ROWS_PER_TILE)],
                      dst_ref.at[pl.ds(r0, ROWS_PER_TILE)])

    @pl.when(s == NS - 1)
    def _():
      pltpu.sync_copy(src_ref.at[pl.ds(r0, LAST_ROWS)],
                      dst_ref.at[pl.ds(r0, LAST_ROWS)])

  @pl.when(c == 0)
  def _():
    stripe_copy(h0_hbm, acc_sh)

  @pl.when(c == 1)
  def _():
    stripe_copy(h1_hbm, acc_sh)

  plsc.subcore_barrier()

  def iload_start(j, k):
    pltpu.async_copy(srcc_hbm.at[pl.ds(ts + j, 1)], src32.at[pl.ds(k, 1)],
                     isems[k])
    pltpu.async_copy(dstc_hbm.at[pl.ds(ts + j, 1)], dst32.at[pl.ds(k, 1)],
                     isems[k])

  def iload_wait(j, k):
    pltpu.make_async_copy(srcc_hbm.at[pl.ds(ts + j, 1)],
                          src32.at[pl.ds(k, 1)], isems[k]).wait()
    pltpu.make_async_copy(dstc_hbm.at[pl.ds(ts + j, 1)],
                          dst32.at[pl.ds(k, 1)], isems[k]).wait()

  def gather_start(k):
    @pl.when(c == 0)
    def _():
      pltpu.async_copy(h0_hbm.at[src32.at[k, 0]], rows_v.at[k], gsems[k])

    @pl.when(c == 1)
    def _():
      pltpu.async_copy(h1_hbm.at[src32.at[k, 0]], rows_v.at[k], gsems[k])

  def gather_wait(k):
    @pl.when(c == 0)
    def _():
      pltpu.make_async_copy(h0_hbm.at[src32.at[k, 0]], rows_v.at[k],
                            gsems[k]).wait()

    @pl.when(c == 1)
    def _():
      pltpu.make_async_copy(h1_hbm.at[src32.at[k, 0]], rows_v.at[k],
                            gsems[k]).wait()

  def scatter_start(k):
    pltpu.async_copy(rows_v.at[k], acc_sh.at[dst32.at[k, 0]], ssems[k],
                     add=True)

  def scatter_wait(k):
    pltpu.make_async_copy(rows_v.at[k], acc_sh.at[dst32.at[k, 0]],
                          ssems[k]).wait()

  n_outer = (nch + D1 + D2 + NBUF - 1) // NBUF

  @pl.loop(0, n_outer)
  def _(t):
    p0 = t * NBUF
    for kk in range(NBUF):
      ji = p0 + kk
      jg = ji - D1
      js = ji - D1 - D2
      kg = (kk - D1) % NBUF
      ks = (kk - D1 - D2) % NBUF

      @pl.when(ji < nch)
      def _():
        @pl.when(ji >= NBUF)
        def _():
          scatter_wait(kk)

        iload_start(ji, kk)

      @pl.when(jnp.logical_and(jg >= 0, jg < nch))
      def _():
        iload_wait(jg, kg)
        gather_start(kg)

      @pl.when(jnp.logical_and(js >= 0, js < nch))
      def _():
        gather_wait(ks)
        scatter_start(ks)

  for k in range(NBUF):
    scatter_wait(k)

  plsc.subcore_barrier()

  @pl.when(c == 0)
  def _():
    stripe_copy(acc_sh, out0_hbm)

  @pl.when(c == 1)
  def _():
    stripe_copy(acc_sh, out1_hbm)


_agg_call = functools.partial(
    pl.kernel,
    out_type=(
        jax.ShapeDtypeStruct((N_PAD, FH), jnp.float32),
        jax.ShapeDtypeStruct((N_PAD, FH), jnp.float32),
    ),
    mesh=plsc.VectorSubcoreMesh(core_axis_name="c", subcore_axis_name="s"),
    scratch_types=[
        pltpu.VMEM((NBUF, 1, CH), jnp.int32),
        pltpu.VMEM((NBUF, 1, CH), jnp.int32),
        pltpu.VMEM((NBUF, CH, FH), jnp.float32),
        pltpu.VMEM_SHARED((N_ACC, FH), jnp.float32),
    ] + [pltpu.SemaphoreType.DMA] * (3 * NBUF),
)(_agg_body)


def _layer_body(u0, u1, w1, b1, w2, b2, gamma, beta, batchb,
                o0, o1, pooled, z_s, ssum_s, ssq_s):
  p = pl.program_id(0)
  i = pl.program_id(1)

  @pl.when(p == 0)
  def _():
    u = jnp.concatenate([u0[...], u1[...]], axis=1)
    a = jnp.dot(u, w1[...], preferred_element_type=jnp.float32) + b1[...]
    a = jnp.maximum(a, 0.0)
    z = jnp.dot(a, w2[...], preferred_element_type=jnp.float32) + b2[...]
    z = jnp.maximum(z, 0.0)
    row = i * BR + lax.broadcasted_iota(jnp.int32, (BR, DIM), 0)
    z = jnp.where(row < N, z, 0.0)
    z_s[pl.ds(i * BR, BR), :] = z
    ssum = jnp.sum(z, axis=0, keepdims=True)
    ssq = jnp.sum(z * z, axis=0, keepdims=True)

    @pl.when(i == 0)
    def _():
      ssum_s[...] = ssum
      ssq_s[...] = ssq

    @pl.when(i > 0)
    def _():
      ssum_s[...] = ssum_s[...] + ssum
      ssq_s[...] = ssq_s[...] + ssq

  @pl.when(p == 1)
  def _():
    mean = ssum_s[...] / N
    var = ssq_s[...] / N - mean * mean
    scale = gamma[...] / jnp.sqrt(var + 1e-5)
    shift = beta[...] - mean * scale
    out = z_s[pl.ds(i * BR, BR), :] * scale + shift
    o0[...] = out[:, :FH]
    o1[...] = out[:, FH:]
    oh = (batchb[0] == lax.broadcasted_iota(jnp.int32, (G, BR), 0)
          ).astype(jnp.float32)
    pc = jnp.dot(oh, out, preferred_element_type=jnp.float32)

    @pl.when(i == 0)
    def _():
      pooled[...] = pc

    @pl.when(i > 0)
    def _():
      pooled[...] = pooled[...] + pc


_layer_call = pl.pallas_call(
    _layer_body,
    grid=(2, NB),
    in_specs=[
        pl.BlockSpec((BR, FH), lambda p, i: (i, 0)),
        pl.BlockSpec((BR, FH), lambda p, i: (i, 0)),
        pl.BlockSpec((DIM, DIM), lambda p, i: (0, 0)),
        pl.BlockSpec((1, DIM), lambda p, i: (0, 0)),
        pl.BlockSpec((DIM, DIM), lambda p, i: (0, 0)),
        pl.BlockSpec((1, DIM), lambda p, i: (0, 0)),
        pl.BlockSpec((1, DIM), lambda p, i: (0, 0)),
        pl.BlockSpec((1, DIM), lambda p, i: (0, 0)),
        pl.BlockSpec((1, 1, BR), lambda p, i: (i, 0, 0)),
    ],
    out_specs=[
        pl.BlockSpec((BR, FH), lambda p, i: (i, 0)),
        pl.BlockSpec((BR, FH), lambda p, i: (i, 0)),
        pl.BlockSpec((G, DIM), lambda p, i: (0, 0)),
    ],
    out_shape=[
        jax.ShapeDtypeStruct((N_PAD, FH), jnp.float32),
        jax.ShapeDtypeStruct((N_PAD, FH), jnp.float32),
        jax.ShapeDtypeStruct((G, DIM), jnp.float32),
    ],
    scratch_shapes=[
        pltpu.VMEM((N_PAD, DIM), jnp.float32),
        pltpu.VMEM((1, DIM), jnp.float32),
        pltpu.VMEM((1, DIM), jnp.float32),
    ],
)


def kernel(x, edge_index, batch, params):
  src = edge_index[0]
  dst = edge_index[1]
  srcc = src.reshape(N_CHUNKS, 1, CH)
  dstc = dst.reshape(N_CHUNKS, 1, CH)
  batch_p = jnp.concatenate(
      [batch, jnp.full((N_PAD - N,), G, jnp.int32)]).reshape(NB, 1, BR)

  xp = jnp.pad(x, ((0, N_PAD - N), (0, 0)))
  h0 = xp[:, :FH]
  h1 = xp[:, FH:]

  pooled_list = []
  xs_list = []
  for p in params:
    u0, u1 = _agg_call(h0, h1, srcc, dstc)
    o0, o1, pooled = _layer_call(
        u0, u1, p["W1"], p["b1"].reshape(1, DIM),
        p["W2"], p["b2"].reshape(1, DIM), p["gamma"].reshape(1, DIM),
        p["beta"].reshape(1, DIM), batch_p)
    h0, h1 = o0, o1
    xs_list.append(jnp.concatenate([o0[:N], o1[:N]], axis=1))
    pooled_list.append(pooled)

  return (jnp.concatenate(pooled_list, axis=1),
          jnp.concatenate(xs_list, axis=1))


# speedup vs baseline: 8.1884x; 1.0795x over previous
import functools

import jax
import jax.numpy as jnp
from jax import lax
from jax.experimental import pallas as pl
from jax.experimental.pallas import tpu as pltpu
from jax.experimental.pallas import tpu_sc as plsc

N = 10000
E = 160000
DIM = 256
G = 64
FH = DIM // 2

N_PAD = 10240

NC, NS = 2, 16
CH = 64
N_CHUNKS = E // CH
CH_BASE = N_CHUNKS // NS
CH_EXTRA = N_CHUNKS % NS
NBUF = 6
D1 = 1
D2 = 4
ROWS_PER_TILE = 624
LAST_ROWS = N - ROWS_PER_TILE * (NS - 1)
N_ACC = N

BR = 1024
NB = N_PAD // BR


def _agg_body(h0_hbm, h1_hbm, srcc_hbm, dstc_hbm, out0_hbm, out1_hbm,
              src32, dst32, rows_v, acc_sh, *sems):
  isems = sems[:NBUF]
  gsems = sems[NBUF:2 * NBUF]
  ssems = sems[2 * NBUF:]
  c = lax.axis_index("c")
  s = lax.axis_index("s")
  r0 = s * ROWS_PER_TILE
  ts = s * CH_BASE + jnp.minimum(s, CH_EXTRA)
  nch = CH_BASE + jnp.where(s < CH_EXTRA, 1, 0)

  def stripe_copy(src_ref, dst_ref):
    @pl.when(s < NS - 1)
    def _():
      pltpu.sync_copy(src_ref.at[pl.ds(r0, ROWS_PER_TILE)],
                      dst_ref.at[pl.ds(r0, ROWS_PER_TILE)])

    @pl.when(s == NS - 1)
    def _():
      pltpu.sync_copy(src_ref.at[pl.ds(r0, LAST_ROWS)],
                      dst_ref.at[pl.ds(r0, LAST_ROWS)])

  @pl.when(c == 0)
  def _():
    stripe_copy(h0_hbm, acc_sh)

  @pl.when(c == 1)
  def _():
    stripe_copy(h1_hbm, acc_sh)

  plsc.subcore_barrier()

  def iload_start(j, k):
    pltpu.async_copy(srcc_hbm.at[pl.ds(ts + j, 1)], src32.at[pl.ds(k, 1)],
                     isems[k])
    pltpu.async_copy(dstc_hbm.at[pl.ds(ts + j, 1)], dst32.at[pl.ds(k, 1)],
                     isems[k])

  def iload_wait(j, k):
    pltpu.make_async_copy(srcc_hbm.at[pl.ds(ts + j, 1)],
                          src32.at[pl.ds(k, 1)], isems[k]).wait()
    pltpu.make_async_copy(dstc_hbm.at[pl.ds(ts + j, 1)],
                          dst32.at[pl.ds(k, 1)], isems[k]).wait()

  def gather_start(k):
    @pl.when(c == 0)
    def _():
      pltpu.async_copy(h0_hbm.at[src32.at[k, 0]], rows_v.at[k], gsems[k])

    @pl.when(c == 1)
    def _():
      pltpu.async_copy(h1_hbm.at[src32.at[k, 0]], rows_v.at[k], gsems[k])

  def gather_wait(k):
    @pl.when(c == 0)
    def _():
      pltpu.make_async_copy(h0_hbm.at[src32.at[k, 0]], rows_v.at[k],
                            gsems[k]).wait()

    @pl.when(c == 1)
    def _():
      pltpu.make_async_copy(h1_hbm.at[src32.at[k, 0]], rows_v.at[k],
                            gsems[k]).wait()

  def scatter_start(k):
    pltpu.async_copy(rows_v.at[k], acc_sh.at[dst32.at[k, 0]], ssems[k],
                     add=True)

  def scatter_wait(k):
    pltpu.make_async_copy(rows_v.at[k], acc_sh.at[dst32.at[k, 0]],
                          ssems[k]).wait()

  n_outer = (nch + D1 + D2 + NBUF - 1) // NBUF

  @pl.loop(0, n_outer)
  def _(t):
    p0 = t * NBUF
    for kk in range(NBUF):
      ji = p0 + kk
      jg = ji - D1
      js = ji - D1 - D2
      kg = (kk - D1) % NBUF
      ks = (kk - D1 - D2) % NBUF

      @pl.when(ji < nch)
      def _():
        @pl.when(ji >= NBUF)
        def _():
          scatter_wait(kk)

        iload_start(ji, kk)

      @pl.when(jnp.logical_and(jg >= 0, jg < nch))
      def _():
        iload_wait(jg, kg)
        gather_start(kg)

      @pl.when(jnp.logical_and(js >= 0, js < nch))
      def _():
        gather_wait(ks)
        scatter_start(ks)

  for k in range(NBUF):
    scatter_wait(k)

  plsc.subcore_barrier()

  @pl.when(c == 0)
  def _():
    stripe_copy(acc_sh, out0_hbm)

  @pl.when(c == 1)
  def _():
    stripe_copy(acc_sh, out1_hbm)


_agg_call = functools.partial(
    pl.kernel,
    out_type=(
        jax.ShapeDtypeStruct((N_PAD, FH), jnp.float32),
        jax.ShapeDtypeStruct((N_PAD, FH), jnp.float32),
    ),
    mesh=plsc.VectorSubcoreMesh(core_axis_name="c", subcore_axis_name="s"),
    scratch_types=[
        pltpu.VMEM((NBUF, 1, CH), jnp.int32),
        pltpu.VMEM((NBUF, 1, CH), jnp.int32),
        pltpu.VMEM((NBUF, CH, FH), jnp.float32),
        pltpu.VMEM_SHARED((N_ACC, FH), jnp.float32),
    ] + [pltpu.SemaphoreType.DMA] * (3 * NBUF),
)(_agg_body)


def _layer_body(u0, u1, w1, b1, w2, b2, gamma, beta, batchb,
                o0, o1, pooled, z_s, ssum_s, ssq_s):
  p = pl.program_id(0)
  i = pl.program_id(1)

  @pl.when(p == 0)
  def _():
    u = jnp.concatenate([u0[...], u1[...]], axis=1)
    a = jnp.dot(u, w1[...], preferred_element_type=jnp.float32) + b1[...]
    a = jnp.maximum(a, 0.0)
    z = jnp.dot(a, w2[...], preferred_element_type=jnp.float32) + b2[...]
    z = jnp.maximum(z, 0.0)
    row = i * BR + lax.broadcasted_iota(jnp.int32, (BR, DIM), 0)
    z = jnp.where(row < N, z, 0.0)
    z_s[pl.ds(i * BR, BR), :] = z
    ssum = jnp.sum(z, axis=0, keepdims=True)
    ssq = jnp.sum(z * z, axis=0, keepdims=True)

    @pl.when(i == 0)
    def _():
      ssum_s[...] = ssum
      ssq_s[...] = ssq

    @pl.when(i > 0)
    def _():
      ssum_s[...] = ssum_s[...] + ssum
      ssq_s[...] = ssq_s[...] + ssq

  @pl.when(p == 1)
  def _():
    mean = ssum_s[...] / N
    var = ssq_s[...] / N - mean * mean
    scale = gamma[...] / jnp.sqrt(var + 1e-5)
    shift = beta[...] - mean * scale
    out = z_s[pl.ds(i * BR, BR), :] * scale + shift
    o0[...] = out[:, :FH]
    o1[...] = out[:, FH:]
    oh = (batchb[0] == lax.broadcasted_iota(jnp.int32, (G, BR), 0)
          ).astype(jnp.float32)
    pc = jnp.dot(oh, out, preferred_element_type=jnp.float32)

    @pl.when(i == 0)
    def _():
      pooled[...] = pc

    @pl.when(i > 0)
    def _():
      pooled[...] = pooled[...] + pc


_layer_call = pl.pallas_call(
    _layer_body,
    grid=(2, NB),
    in_specs=[
        pl.BlockSpec((BR, FH), lambda p, i: (i, 0)),
        pl.BlockSpec((BR, FH), lambda p, i: (i, 0)),
        pl.BlockSpec((DIM, DIM), lambda p, i: (0, 0)),
        pl.BlockSpec((1, DIM), lambda p, i: (0, 0)),
        pl.BlockSpec((DIM, DIM), lambda p, i: (0, 0)),
        pl.BlockSpec((1, DIM), lambda p, i: (0, 0)),
        pl.BlockSpec((1, DIM), lambda p, i: (0, 0)),
        pl.BlockSpec((1, DIM), lambda p, i: (0, 0)),
        pl.BlockSpec((1, 1, BR), lambda p, i: (i, 0, 0)),
    ],
    out_specs=[
        pl.BlockSpec((BR, FH), lambda p, i: (i, 0)),
        pl.BlockSpec((BR, FH), lambda p, i: (i, 0)),
        pl.BlockSpec((G, DIM), lambda p, i: (0, 0)),
    ],
    out_shape=[
        jax.ShapeDtypeStruct((N_PAD, FH), jnp.float32),
        jax.ShapeDtypeStruct((N_PAD, FH), jnp.float32),
        jax.ShapeDtypeStruct((G, DIM), jnp.float32),
    ],
    scratch_shapes=[
        pltpu.VMEM((N_PAD, DIM), jnp.float32),
        pltpu.VMEM((1, DIM), jnp.float32),
        pltpu.VMEM((1, DIM), jnp.float32),
    ],
)


def kernel(x, edge_index, batch, params):
  src = edge_index[0]
  dst = edge_index[1]
  srcc = src.reshape(N_CHUNKS, 1, CH)
  dstc = dst.reshape(N_CHUNKS, 1, CH)
  batch_p = jnp.concatenate(
      [batch, jnp.full((N_PAD - N,), G, jnp.int32)]).reshape(NB, 1, BR)

  xp = jnp.pad(x, ((0, N_PAD - N), (0, 0)))
  h0 = xp[:, :FH]
  h1 = xp[:, FH:]

  pooled_list = []
  xs_list = []
  for p in params:
    u0, u1 = _agg_call(h0, h1, srcc, dstc)
    o0, o1, pooled = _layer_call(
        u0, u1, p["W1"], p["b1"].reshape(1, DIM),
        p["W2"], p["b2"].reshape(1, DIM), p["gamma"].reshape(1, DIM),
        p["beta"].reshape(1, DIM), batch_p)
    h0, h1 = o0, o1
    xs_list.append(jnp.concatenate([o0[:N], o1[:N]], axis=1))
    pooled_list.append(pooled)

  return (jnp.concatenate(pooled_list, axis=1),
          jnp.concatenate(xs_list, axis=1))


# speedup vs baseline: 8.5419x; 1.0432x over previous
import functools

import jax
import jax.numpy as jnp
from jax import lax
from jax.experimental import pallas as pl
from jax.experimental.pallas import tpu as pltpu
from jax.experimental.pallas import tpu_sc as plsc

N = 10000
E = 160000
DIM = 256
G = 64
FH = DIM // 2

N_PAD = 10240

NC, NS = 2, 16
CH = 64
N_CHUNKS = E // CH
CH_BASE = N_CHUNKS // NS
CH_EXTRA = N_CHUNKS % NS
NBUF = 6
D1 = 1
D2 = 4
ROWS_PER_TILE = 624
LAST_ROWS = N - ROWS_PER_TILE * (NS - 1)
N_ACC = N

BR = 2048
NB = N_PAD // BR


def _agg_body(h0_hbm, h1_hbm, srcc_hbm, dstc_hbm, out0_hbm, out1_hbm,
              src32, dst32, rows_v, acc_sh, *sems):
  isems = sems[:NBUF]
  gsems = sems[NBUF:2 * NBUF]
  ssems = sems[2 * NBUF:]
  c = lax.axis_index("c")
  s = lax.axis_index("s")
  r0 = s * ROWS_PER_TILE
  ts = s * CH_BASE + jnp.minimum(s, CH_EXTRA)
  nch = CH_BASE + jnp.where(s < CH_EXTRA, 1, 0)

  def stripe_copy(src_ref, dst_ref):
    @pl.when(s < NS - 1)
    def _():
      pltpu.sync_copy(src_ref.at[pl.ds(r0, ROWS_PER_TILE)],
                      dst_ref.at[pl.ds(r0, ROWS_PER_TILE)])

    @pl.when(s == NS - 1)
    def _():
      pltpu.sync_copy(src_ref.at[pl.ds(r0, LAST_ROWS)],
                      dst_ref.at[pl.ds(r0, LAST_ROWS)])

  @pl.when(c == 0)
  def _():
    stripe_copy(h0_hbm, acc_sh)

  @pl.when(c == 1)
  def _():
    stripe_copy(h1_hbm, acc_sh)

  plsc.subcore_barrier()

  def iload_start(j, k):
    pltpu.async_copy(srcc_hbm.at[pl.ds(ts + j, 1)], src32.at[pl.ds(k, 1)],
                     isems[k])
    pltpu.async_copy(dstc_hbm.at[pl.ds(ts + j, 1)], dst32.at[pl.ds(k, 1)],
                     isems[k])

  def iload_wait(j, k):
    pltpu.make_async_copy(srcc_hbm.at[pl.ds(ts + j, 1)],
                          src32.at[pl.ds(k, 1)], isems[k]).wait()
    pltpu.make_async_copy(dstc_hbm.at[pl.ds(ts + j, 1)],
                          dst32.at[pl.ds(k, 1)], isems[k]).wait()

  def gather_start(k):
    @pl.when(c == 0)
    def _():
      pltpu.async_copy(h0_hbm.at[src32.at[k, 0]], rows_v.at[k], gsems[k])

    @pl.when(c == 1)
    def _():
      pltpu.async_copy(h1_hbm.at[src32.at[k, 0]], rows_v.at[k], gsems[k])

  def gather_wait(k):
    @pl.when(c == 0)
    def _():
      pltpu.make_async_copy(h0_hbm.at[src32.at[k, 0]], rows_v.at[k],
                            gsems[k]).wait()

    @pl.when(c == 1)
    def _():
      pltpu.make_async_copy(h1_hbm.at[src32.at[k, 0]], rows_v.at[k],
                            gsems[k]).wait()

  def scatter_start(k):
    pltpu.async_copy(rows_v.at[k], acc_sh.at[dst32.at[k, 0]], ssems[k],
                     add=True)

  def scatter_wait(k):
    pltpu.make_async_copy(rows_v.at[k], acc_sh.at[dst32.at[k, 0]],
                          ssems[k]).wait()

  n_outer = (nch + D1 + D2 + NBUF - 1) // NBUF

  @pl.loop(0, n_outer)
  def _(t):
    p0 = t * NBUF
    for kk in range(NBUF):
      ji = p0 + kk
      jg = ji - D1
      js = ji - D1 - D2
      kg = (kk - D1) % NBUF
      ks = (kk - D1 - D2) % NBUF

      @pl.when(ji < nch)
      def _():
        @pl.when(ji >= NBUF)
        def _():
          scatter_wait(kk)

        iload_start(ji, kk)

      @pl.when(jnp.logical_and(jg >= 0, jg < nch))
      def _():
        iload_wait(jg, kg)
        gather_start(kg)

      @pl.when(jnp.logical_and(js >= 0, js < nch))
      def _():
        gather_wait(ks)
        scatter_start(ks)

  for k in range(NBUF):
    scatter_wait(k)

  plsc.subcore_barrier()

  @pl.when(c == 0)
  def _():
    stripe_copy(acc_sh, out0_hbm)

  @pl.when(c == 1)
  def _():
    stripe_copy(acc_sh, out1_hbm)


_agg_call = functools.partial(
    pl.kernel,
    out_type=(
        jax.ShapeDtypeStruct((N_PAD, FH), jnp.float32),
        jax.ShapeDtypeStruct((N_PAD, FH), jnp.float32),
    ),
    mesh=plsc.VectorSubcoreMesh(core_axis_name="c", subcore_axis_name="s"),
    scratch_types=[
        pltpu.VMEM((NBUF, 1, CH), jnp.int32),
        pltpu.VMEM((NBUF, 1, CH), jnp.int32),
        pltpu.VMEM((NBUF, CH, FH), jnp.float32),
        pltpu.VMEM_SHARED((N_ACC, FH), jnp.float32),
    ] + [pltpu.SemaphoreType.DMA] * (3 * NBUF),
)(_agg_body)


def _layer_body(u0, u1, w1, b1, w2, b2, gamma, beta, batchb,
                o0, o1, pooled, z_s, ssum_s, ssq_s):
  p = pl.program_id(0)
  i = pl.program_id(1)

  @pl.when(p == 0)
  def _():
    u = jnp.concatenate([u0[...], u1[...]], axis=1)
    a = jnp.dot(u, w1[...], preferred_element_type=jnp.float32) + b1[...]
    a = jnp.maximum(a, 0.0)
    z = jnp.dot(a, w2[...], preferred_element_type=jnp.float32) + b2[...]
    z = jnp.maximum(z, 0.0)
    row = i * BR + lax.broadcasted_iota(jnp.int32, (BR, DIM), 0)
    z = jnp.where(row < N, z, 0.0)
    z_s[pl.ds(i * BR, BR), :] = z
    ssum = jnp.sum(z, axis=0, keepdims=True)
    ssq = jnp.sum(z * z, axis=0, keepdims=True)

    @pl.when(i == 0)
    def _():
      ssum_s[...] = ssum
      ssq_s[...] = ssq

    @pl.when(i > 0)
    def _():
      ssum_s[...] = ssum_s[...] + ssum
      ssq_s[...] = ssq_s[...] + ssq

  @pl.when(p == 1)
  def _():
    mean = ssum_s[...] / N
    var = ssq_s[...] / N - mean * mean
    scale = gamma[...] / jnp.sqrt(var + 1e-5)
    shift = beta[...] - mean * scale
    out = z_s[pl.ds(i * BR, BR), :] * scale + shift
    o0[...] = out[:, :FH]
    o1[...] = out[:, FH:]
    oh = (batchb[0] == lax.broadcasted_iota(jnp.int32, (G, BR), 0)
          ).astype(jnp.float32)
    pc = jnp.dot(oh, out, preferred_element_type=jnp.float32)

    @pl.when(i == 0)
    def _():
      pooled[...] = pc

    @pl.when(i > 0)
    def _():
      pooled[...] = pooled[...] + pc


_layer_call = pl.pallas_call(
    _layer_body,
    grid=(2, NB),
    in_specs=[
        pl.BlockSpec((BR, FH), lambda p, i: (i, 0)),
        pl.BlockSpec((BR, FH), lambda p, i: (i, 0)),
        pl.BlockSpec((DIM, DIM), lambda p, i: (0, 0)),
        pl.BlockSpec((1, DIM), lambda p, i: (0, 0)),
        pl.BlockSpec((DIM, DIM), lambda p, i: (0, 0)),
        pl.BlockSpec((1, DIM), lambda p, i: (0, 0)),
        pl.BlockSpec((1, DIM), lambda p, i: (0, 0)),
        pl.BlockSpec((1, DIM), lambda p, i: (0, 0)),
        pl.BlockSpec((1, 1, BR), lambda p, i: (i, 0, 0)),
    ],
    out_specs=[
        pl.BlockSpec((BR, FH), lambda p, i: (i, 0)),
        pl.BlockSpec((BR, FH), lambda p, i: (i, 0)),
        pl.BlockSpec((G, DIM), lambda p, i: (0, 0)),
    ],
    out_shape=[
        jax.ShapeDtypeStruct((N_PAD, FH), jnp.float32),
        jax.ShapeDtypeStruct((N_PAD, FH), jnp.float32),
        jax.ShapeDtypeStruct((G, DIM), jnp.float32),
    ],
    scratch_shapes=[
        pltpu.VMEM((N_PAD, DIM), jnp.float32),
        pltpu.VMEM((1, DIM), jnp.float32),
        pltpu.VMEM((1, DIM), jnp.float32),
    ],
)


def kernel(x, edge_index, batch, params):
  src = edge_index[0]
  dst = edge_index[1]
  srcc = src.reshape(N_CHUNKS, 1, CH)
  dstc = dst.reshape(N_CHUNKS, 1, CH)
  batch_p = jnp.concatenate(
      [batch, jnp.full((N_PAD - N,), G, jnp.int32)]).reshape(NB, 1, BR)

  xp = jnp.pad(x, ((0, N_PAD - N), (0, 0)))
  h0 = xp[:, :FH]
  h1 = xp[:, FH:]

  pooled_list = []
  xs_list = []
  for p in params:
    u0, u1 = _agg_call(h0, h1, srcc, dstc)
    o0, o1, pooled = _layer_call(
        u0, u1, p["W1"], p["b1"].reshape(1, DIM),
        p["W2"], p["b2"].reshape(1, DIM), p["gamma"].reshape(1, DIM),
        p["beta"].reshape(1, DIM), batch_p)
    h0, h1 = o0, o1
    xs_list.append(jnp.concatenate([o0[:N], o1[:N]], axis=1))
    pooled_list.append(pooled)

  return (jnp.concatenate(pooled_list, axis=1),
          jnp.concatenate(xs_list, axis=1))


# speedup vs baseline: 8.5653x; 1.0027x over previous
import functools

import jax
import jax.numpy as jnp
from jax import lax
from jax.experimental import pallas as pl
from jax.experimental.pallas import tpu as pltpu
from jax.experimental.pallas import tpu_sc as plsc

N = 10000
E = 160000
DIM = 256
G = 64
FH = DIM // 2

N_PAD = 10240

NC, NS = 2, 16
CH = 64
N_CHUNKS = E // CH
CH_BASE = N_CHUNKS // NS
CH_EXTRA = N_CHUNKS % NS
NBUF = 6
D1 = 1
D2 = 4
ROWS_PER_TILE = 624
LAST_ROWS = N - ROWS_PER_TILE * (NS - 1)
N_ACC = N

BR = 2560
NB = N_PAD // BR


def _agg_body(h0_hbm, h1_hbm, srcc_hbm, dstc_hbm, out0_hbm, out1_hbm,
              src32, dst32, rows_v, acc_sh, *sems):
  isems = sems[:NBUF]
  gsems = sems[NBUF:2 * NBUF]
  ssems = sems[2 * NBUF:]
  c = lax.axis_index("c")
  s = lax.axis_index("s")
  r0 = s * ROWS_PER_TILE
  ts = s * CH_BASE + jnp.minimum(s, CH_EXTRA)
  nch = CH_BASE + jnp.where(s < CH_EXTRA, 1, 0)

  def stripe_copy(src_ref, dst_ref):
    @pl.when(s < NS - 1)
    def _():
      pltpu.sync_copy(src_ref.at[pl.ds(r0, ROWS_PER_TILE)],
                      dst_ref.at[pl.ds(r0, ROWS_PER_TILE)])

    @pl.when(s == NS - 1)
    def _():
      pltpu.sync_copy(src_ref.at[pl.ds(r0, LAST_ROWS)],
                      dst_ref.at[pl.ds(r0, LAST_ROWS)])

  @pl.when(c == 0)
  def _():
    stripe_copy(h0_hbm, acc_sh)

  @pl.when(c == 1)
  def _():
    stripe_copy(h1_hbm, acc_sh)

  plsc.subcore_barrier()

  def iload_start(j, k):
    pltpu.async_copy(srcc_hbm.at[pl.ds(ts + j, 1)], src32.at[pl.ds(k, 1)],
                     isems[k])
    pltpu.async_copy(dstc_hbm.at[pl.ds(ts + j, 1)], dst32.at[pl.ds(k, 1)],
                     isems[k])

  def iload_wait(j, k):
    pltpu.make_async_copy(srcc_hbm.at[pl.ds(ts + j, 1)],
                          src32.at[pl.ds(k, 1)], isems[k]).wait()
    pltpu.make_async_copy(dstc_hbm.at[pl.ds(ts + j, 1)],
                          dst32.at[pl.ds(k, 1)], isems[k]).wait()

  def gather_start(k):
    @pl.when(c == 0)
    def _():
      pltpu.async_copy(h0_hbm.at[src32.at[k, 0]], rows_v.at[k], gsems[k])

    @pl.when(c == 1)
    def _():
      pltpu.async_copy(h1_hbm.at[src32.at[k, 0]], rows_v.at[k], gsems[k])

  def gather_wait(k):
    @pl.when(c == 0)
    def _():
      pltpu.make_async_copy(h0_hbm.at[src32.at[k, 0]], rows_v.at[k],
                            gsems[k]).wait()

    @pl.when(c == 1)
    def _():
      pltpu.make_async_copy(h1_hbm.at[src32.at[k, 0]], rows_v.at[k],
                            gsems[k]).wait()

  def scatter_start(k):
    pltpu.async_copy(rows_v.at[k], acc_sh.at[dst32.at[k, 0]], ssems[k],
                     add=True)

  def scatter_wait(k):
    pltpu.make_async_copy(rows_v.at[k], acc_sh.at[dst32.at[k, 0]],
                          ssems[k]).wait()

  n_outer = (nch + D1 + D2 + NBUF - 1) // NBUF

  @pl.loop(0, n_outer)
  def _(t):
    p0 = t * NBUF
    for kk in range(NBUF):
      ji = p0 + kk
      jg = ji - D1
      js = ji - D1 - D2
      kg = (kk - D1) % NBUF
      ks = (kk - D1 - D2) % NBUF

      @pl.when(ji < nch)
      def _():
        @pl.when(ji >= NBUF)
        def _():
          scatter_wait(kk)

        iload_start(ji, kk)

      @pl.when(jnp.logical_and(jg >= 0, jg < nch))
      def _():
        iload_wait(jg, kg)
        gather_start(kg)

      @pl.when(jnp.logical_and(js >= 0, js < nch))
      def _():
        gather_wait(ks)
        scatter_start(ks)

  for k in range(NBUF):
    scatter_wait(k)

  plsc.subcore_barrier()

  @pl.when(c == 0)
  def _():
    stripe_copy(acc_sh, out0_hbm)

  @pl.when(c == 1)
  def _():
    stripe_copy(acc_sh, out1_hbm)


_agg_call = functools.partial(
    pl.kernel,
    out_type=(
        jax.ShapeDtypeStruct((N_PAD, FH), jnp.float32),
        jax.ShapeDtypeStruct((N_PAD, FH), jnp.float32),
    ),
    mesh=plsc.VectorSubcoreMesh(core_axis_name="c", subcore_axis_name="s"),
    scratch_types=[
        pltpu.VMEM((NBUF, 1, CH), jnp.int32),
        pltpu.VMEM((NBUF, 1, CH), jnp.int32),
        pltpu.VMEM((NBUF, CH, FH), jnp.float32),
        pltpu.VMEM_SHARED((N_ACC, FH), jnp.float32),
    ] + [pltpu.SemaphoreType.DMA] * (3 * NBUF),
)(_agg_body)


def _layer_body(u0, u1, w1, b1, w2, b2, gamma, beta, batchb,
                o0, o1, pooled, z_s, ssum_s, ssq_s):
  p = pl.program_id(0)
  i = pl.program_id(1)

  @pl.when(p == 0)
  def _():
    u = jnp.concatenate([u0[...], u1[...]], axis=1)
    a = jnp.dot(u, w1[...], preferred_element_type=jnp.float32) + b1[...]
    a = jnp.maximum(a, 0.0)
    z = jnp.dot(a, w2[...], preferred_element_type=jnp.float32) + b2[...]
    z = jnp.maximum(z, 0.0)
    row = i * BR + lax.broadcasted_iota(jnp.int32, (BR, DIM), 0)
    z = jnp.where(row < N, z, 0.0)
    z_s[pl.ds(i * BR, BR), :] = z
    ssum = jnp.sum(z, axis=0, keepdims=True)
    ssq = jnp.sum(z * z, axis=0, keepdims=True)

    @pl.when(i == 0)
    def _():
      ssum_s[...] = ssum
      ssq_s[...] = ssq

    @pl.when(i > 0)
    def _():
      ssum_s[...] = ssum_s[...] + ssum
      ssq_s[...] = ssq_s[...] + ssq

  @pl.when(p == 1)
  def _():
    mean = ssum_s[...] / N
    var = ssq_s[...] / N - mean * mean
    scale = gamma[...] / jnp.sqrt(var + 1e-5)
    shift = beta[...] - mean * scale
    out = z_s[pl.ds(i * BR, BR), :] * scale + shift
    o0[...] = out[:, :FH]
    o1[...] = out[:, FH:]
    oh = (batchb[0] == lax.broadcasted_iota(jnp.int32, (G, BR), 0)
          ).astype(jnp.float32)
    pc = jnp.dot(oh, out, preferred_element_type=jnp.float32)

    @pl.when(i == 0)
    def _():
      pooled[...] = pc

    @pl.when(i > 0)
    def _():
      pooled[...] = pooled[...] + pc


_layer_call = pl.pallas_call(
    _layer_body,
    grid=(2, NB),
    in_specs=[
        pl.BlockSpec((BR, FH), lambda p, i: (i, 0)),
        pl.BlockSpec((BR, FH), lambda p, i: (i, 0)),
        pl.BlockSpec((DIM, DIM), lambda p, i: (0, 0)),
        pl.BlockSpec((1, DIM), lambda p, i: (0, 0)),
        pl.BlockSpec((DIM, DIM), lambda p, i: (0, 0)),
        pl.BlockSpec((1, DIM), lambda p, i: (0, 0)),
        pl.BlockSpec((1, DIM), lambda p, i: (0, 0)),
        pl.BlockSpec((1, DIM), lambda p, i: (0, 0)),
        pl.BlockSpec((1, 1, BR), lambda p, i: (i, 0, 0)),
    ],
    out_specs=[
        pl.BlockSpec((BR, FH), lambda p, i: (i, 0)),
        pl.BlockSpec((BR, FH), lambda p, i: (i, 0)),
        pl.BlockSpec((G, DIM), lambda p, i: (0, 0)),
    ],
    out_shape=[
        jax.ShapeDtypeStruct((N_PAD, FH), jnp.float32),
        jax.ShapeDtypeStruct((N_PAD, FH), jnp.float32),
        jax.ShapeDtypeStruct((G, DIM), jnp.float32),
    ],
    scratch_shapes=[
        pltpu.VMEM((N_PAD, DIM), jnp.float32),
        pltpu.VMEM((1, DIM), jnp.float32),
        pltpu.VMEM((1, DIM), jnp.float32),
    ],
)


def kernel(x, edge_index, batch, params):
  src = edge_index[0]
  dst = edge_index[1]
  srcc = src.reshape(N_CHUNKS, 1, CH)
  dstc = dst.reshape(N_CHUNKS, 1, CH)
  batch_p = jnp.concatenate(
      [batch, jnp.full((N_PAD - N,), G, jnp.int32)]).reshape(NB, 1, BR)

  xp = jnp.pad(x, ((0, N_PAD - N), (0, 0)))
  h0 = xp[:, :FH]
  h1 = xp[:, FH:]

  pooled_list = []
  xs_list = []
  for p in params:
    u0, u1 = _agg_call(h0, h1, srcc, dstc)
    o0, o1, pooled = _layer_call(
        u0, u1, p["W1"], p["b1"].reshape(1, DIM),
        p["W2"], p["b2"].reshape(1, DIM), p["gamma"].reshape(1, DIM),
        p["beta"].reshape(1, DIM), batch_p)
    h0, h1 = o0, o1
    xs_list.append(jnp.concatenate([o0[:N], o1[:N]], axis=1))
    pooled_list.append(pooled)

  return (jnp.concatenate(pooled_list, axis=1),
          jnp.concatenate(xs_list, axis=1))
